# Initial kernel scaffold; baseline (speedup 1.0000x reference)
import jax
import jax.numpy as jnp
from jax.experimental import pallas as pl


def kernel(x, params):
    raise NotImplementedError("write your pallas kernel here")


# speedup vs baseline: 1.1249x; 1.1249x over previous
import functools

import jax
import jax.numpy as jnp
from jax.experimental import pallas as pl

_EPS_BN = 1e-5


def _mlp_body(*refs, preact, stats):
    if preact:
        x_ref, w_ref, b_ref, s_ref, t_ref = refs[:5]
        refs = refs[5:]
    else:
        x_ref, w_ref, b_ref = refs[:3]
        refs = refs[3:]
    y_ref = refs[0]
    i = pl.program_id(0)
    x = x_ref[...]
    if preact:
        x = jnp.maximum(x * s_ref[...] + t_ref[...], 0.0)
    y = jnp.dot(x, w_ref[...], preferred_element_type=jnp.float32) + b_ref[...]
    y_ref[...] = y
    if stats:
        stats_ref = refs[1]

        @pl.when(i == 0)
        def _():
            stats_ref[...] = jnp.zeros_like(stats_ref)

        stats_ref[0:1, :] += jnp.sum(y, axis=0, keepdims=True)
        stats_ref[1:2, :] += jnp.sum(y * y, axis=0, keepdims=True)


def _mlp(x, w, b, s, t, *, preact=True, stats=True):
    m, k = x.shape
    n = w.shape[1]
    bm = min(512, m)
    grid = (m // bm,)
    in_specs = [
        pl.BlockSpec((bm, k), lambda i: (i, 0)),
        pl.BlockSpec((k, n), lambda i: (0, 0)),
        pl.BlockSpec((1, n), lambda i: (0, 0)),
    ]
    args = [x, w, b.reshape(1, n)]
    if preact:
        in_specs += [pl.BlockSpec((1, k), lambda i: (0, 0)),
                     pl.BlockSpec((1, k), lambda i: (0, 0))]
        args += [s.reshape(1, k), t.reshape(1, k)]
    out_shape = [jax.ShapeDtypeStruct((m, n), jnp.float32)]
    out_specs = [pl.BlockSpec((bm, n), lambda i: (i, 0))]
    if stats:
        out_shape.append(jax.ShapeDtypeStruct((8, n), jnp.float32))
        out_specs.append(pl.BlockSpec((8, n), lambda i: (0, 0)))
    body = functools.partial(_mlp_body, preact=preact, stats=stats)
    res = pl.pallas_call(
        body, grid=grid, in_specs=in_specs, out_specs=out_specs,
        out_shape=out_shape)(*args)
    return res if stats else res[0]


def _expert_body(h_ref, w1_ref, c_ref, o_ref, st_ref, *, e_total):
    i = pl.program_id(0)
    j = pl.program_id(1)
    h = h_ref[...]
    eo = jnp.maximum(
        jnp.dot(h, w1_ref[0], preferred_element_type=jnp.float32), 0.0)
    lane = jax.lax.broadcasted_iota(jnp.int32, (1, c_ref.shape[1]), 1)
    sel = jnp.sum(c_ref[...] * (lane == j).astype(jnp.float32), axis=1,
                  keepdims=True)

    @pl.when(j == 0)
    def _():
        o_ref[...] = h

    o_ref[...] += sel * eo

    @pl.when(j == e_total - 1)
    def _():
        @pl.when(i == 0)
        def _():
            st_ref[...] = jnp.zeros_like(st_ref)

        o = o_ref[...]
        st_ref[0:1, :] += jnp.sum(o, axis=0, keepdims=True)
        st_ref[1:2, :] += jnp.sum(o * o, axis=0, keepdims=True)


def _expert_stats(h, w1, combine):
    m, n = h.shape
    e = w1.shape[0]
    bm = min(1024, m)
    body = functools.partial(_expert_body, e_total=e)
    return pl.pallas_call(
        body, grid=(m // bm, e),
        in_specs=[
            pl.BlockSpec((bm, n), lambda i, j: (i, 0)),
            pl.BlockSpec((1, n, n), lambda i, j: (j, 0, 0)),
            pl.BlockSpec((bm, e), lambda i, j: (i, 0)),
        ],
        out_specs=[
            pl.BlockSpec((bm, n), lambda i, j: (i, 0)),
            pl.BlockSpec((8, n), lambda i, j: (0, 0)),
        ],
        out_shape=[
            jax.ShapeDtypeStruct((m, n), jnp.float32),
            jax.ShapeDtypeStruct((8, n), jnp.float32),
        ])(h, w1, combine)


def _affine(stats, g, beta, m):
    su, sq = stats[0], stats[1]
    mu = su / m
    var = sq / m - mu * mu
    s = g / jnp.sqrt(var + _EPS_BN)
    t = beta - mu * s
    return s, t


def _bn_expr(x, g, b):
    mu = jnp.mean(x, axis=0)
    var = jnp.var(x, axis=0)
    return g * (x - mu) / jnp.sqrt(var + _EPS_BN) + b


def _router(h, sp, m):
    hh = h @ sp['W'] + sp['b']
    logits = hh @ sp['gate']
    e = sp['gate'].shape[1]
    gates = jax.nn.softmax(logits, axis=-1)
    top2_vals, top2_idx = jax.lax.top_k(gates, 2)
    denom = jnp.sum(top2_vals, axis=-1, keepdims=True) + 1e-9
    top2_w = top2_vals / denom
    combine = jnp.zeros_like(gates).at[
        jnp.arange(m)[:, None], top2_idx].set(top2_w)
    one_hot_top1 = jax.nn.one_hot(top2_idx[:, 0], e, dtype=jnp.float32)
    density = jnp.mean(one_hot_top1, axis=0)
    density_proxy = jnp.mean(gates, axis=0)
    aux = jnp.mean(density * density_proxy) * float(e * e)
    return hh, combine, aux


def kernel(x, params):
    p = params
    m = x.shape[0]
    h = x @ p['in_W'] + p['in_b']
    h = jax.nn.relu(_bn_expr(h, p['in_g'], p['in_beta']))
    loss = jnp.float32(0.0)
    for sp in p['sparse'][:-1]:
        hh, combine, aux = _router(h, sp, m)
        loss = loss + aux
        expert_out = jax.nn.relu(jnp.einsum('nd,edh->enh', hh, sp['w1']))
        moe_out = jnp.einsum('ne,enh->nh', combine, expert_out)
        out = hh + moe_out
        h = jax.nn.relu(_bn_expr(out, sp['g'], sp['beta']))
    sp = p['sparse'][-1]
    hh, combine, aux = _router(h, sp, m)
    loss = loss + aux
    y, st = _expert_stats(hh, sp['w1'], combine)
    s, t = _affine(st, sp['g'], sp['beta'], m)
    for dp in p['dense']:
        y, st = _mlp(y, dp['W'], dp['b'], s, t)
        s, t = _affine(st, dp['g'], dp['beta'], m)
    out = _mlp(y, p['out_W'], p['out_b'], s, t, preact=True, stats=False)
    return out, loss
